# baseline (speedup 1.0000x reference)
import functools

import jax
import jax.numpy as jnp
from jax import lax
from jax.experimental import pallas as pl
from jax.experimental.pallas import tpu as pltpu
from jax.experimental.pallas import tpu_sc as plsc

_NW = 32
_CHUNK = 128


def _sc_gather(labels, table):
    B = labels.shape[0]
    D = table.shape[1]
    b_per_w = B // _NW

    mesh = plsc.VectorSubcoreMesh(core_axis_name="c", subcore_axis_name="s")

    @functools.partial(
        pl.kernel,
        out_type=jax.ShapeDtypeStruct((B, D), jnp.float32),
        mesh=mesh,
        scratch_types=[
            pltpu.VMEM((b_per_w,), jnp.int32),
            pltpu.VMEM((b_per_w, D), jnp.float32),
            pltpu.SemaphoreType.DMA,
        ],
    )
    def gather_kernel(labels_hbm, table_hbm, out_hbm, idx_v, rows_v, sem):
        wid = lax.axis_index("s") * 2 + lax.axis_index("c")
        base = wid * b_per_w
        pltpu.sync_copy(labels_hbm.at[pl.ds(base, b_per_w)], idx_v)

        def body(c, carry):
            vec = idx_v[pl.ds(c * 16, 16)]
            for l in range(16):
                i = vec[l]
                pltpu.make_async_copy(
                    table_hbm.at[pl.ds(i, 1)],
                    rows_v.at[pl.ds(c * 16 + l, 1)],
                    sem,
                ).start()
            return carry

        lax.fori_loop(0, b_per_w // 16, body, 0)
        pltpu.make_async_copy(
            table_hbm.at[pl.ds(0, b_per_w)], rows_v, sem
        ).wait()
        pltpu.sync_copy(rows_v, out_hbm.at[pl.ds(base, b_per_w)])

    return gather_kernel(labels, table)


def _ln_head_body(x_ref, g_ref, bt_ref, w_ref, bb_ref, y_ref, u_ref):
    x = x_ref[...]
    mean = jnp.mean(x, axis=-1, keepdims=True)
    xc = x - mean
    var = jnp.mean(xc * xc, axis=-1, keepdims=True)
    inv = lax.rsqrt(var + 1e-5)
    y = xc * inv * g_ref[...] + bt_ref[...]
    y_ref[...] = y
    z = jnp.sum(y * w_ref[...], axis=-1, keepdims=True) + bb_ref[...]
    u_ref[...] = jax.nn.sigmoid(z)


def _tc_ln_head(rows, gamma, beta, W, b):
    B, D = rows.shape
    BLK = 2048
    grid = (B // BLK,)
    g2 = gamma.reshape(1, D)
    bt2 = beta.reshape(1, D)
    w2 = W.reshape(1, D)
    b2 = b.reshape(1, 1)
    return pl.pallas_call(
        _ln_head_body,
        grid=grid,
        in_specs=[
            pl.BlockSpec((BLK, D), lambda i: (i, 0)),
            pl.BlockSpec((1, D), lambda i: (0, 0)),
            pl.BlockSpec((1, D), lambda i: (0, 0)),
            pl.BlockSpec((1, D), lambda i: (0, 0)),
            pl.BlockSpec((1, 1), lambda i: (0, 0)),
        ],
        out_specs=[
            pl.BlockSpec((BLK, D), lambda i: (i, 0)),
            pl.BlockSpec((BLK, 1), lambda i: (i, 0)),
        ],
        out_shape=[
            jax.ShapeDtypeStruct((B, D), jnp.float32),
            jax.ShapeDtypeStruct((B, 1), jnp.float32),
        ],
    )(rows, g2, bt2, w2, b2)


@jax.jit
def kernel(class_labels, emb_table, ln_gamma, ln_beta, unc_W, unc_b):
    rows = _sc_gather(class_labels.astype(jnp.int32), emb_table)
    mean = jnp.mean(rows, axis=-1, keepdims=True)
    xc = rows - mean
    var = jnp.mean(xc * xc, axis=-1, keepdims=True)
    y = xc * lax.rsqrt(var + 1e-5) * ln_gamma + ln_beta
    u = jax.nn.sigmoid(y @ unc_W.T + unc_b)
    return (y, u)


# speedup vs baseline: 1.4205x; 1.4205x over previous
import functools

import jax
import jax.numpy as jnp
from jax import lax
from jax.experimental import pallas as pl
from jax.experimental.pallas import tpu as pltpu
from jax.experimental.pallas import tpu_sc as plsc

_NW = 32
_L = 16
_CW = 512


def _sc_stream_extract(sorted_labels, table_t, table_tail):
    D, V = table_t.shape
    B = sorted_labels.shape[0]
    b_per_w = B // _NW
    n_full = V // _CW
    tail_w = V - n_full * _CW
    assert table_tail.shape == (D * tail_w,)
    mesh = plsc.VectorSubcoreMesh(core_axis_name="c", subcore_axis_name="s")

    @functools.partial(
        pl.kernel,
        out_type=jax.ShapeDtypeStruct((D, B), jnp.float32),
        mesh=mesh,
        compiler_params=pltpu.CompilerParams(needs_layout_passes=False),
        scratch_types=[
            pltpu.VMEM((b_per_w + 2 * _L,), jnp.int32),
            pltpu.VMEM((2, D, _CW), jnp.float32),
            pltpu.VMEM((D * tail_w,), jnp.float32),
            pltpu.VMEM((D, b_per_w), jnp.float32),
            pltpu.SemaphoreType.DMA,
            pltpu.SemaphoreType.DMA,
        ],
    )
    def stream_kernel(sorted_hbm, table_hbm, tail_hbm, out_hbm,
                      sl_v, staged_v, tail_v, rows_v, sem0, sem1):
        wid = lax.axis_index("s") * 2 + lax.axis_index("c")
        base = wid * b_per_w
        pltpu.sync_copy(sorted_hbm.at[pl.ds(base, b_per_w)],
                        sl_v.at[pl.ds(0, b_per_w)])
        pltpu.sync_copy(tail_hbm, tail_v)
        big = jnp.full((_L,), jnp.int32(2**30), jnp.int32)
        sl_v[pl.ds(b_per_w, _L)] = big
        sl_v[pl.ds(b_per_w + _L, _L)] = big

        v_lo = lax.reduce_min(sl_v[pl.ds(0, _L)], (0,))
        v_hi = lax.reduce_max(sl_v[pl.ds(b_per_w - _L, _L)], (0,))
        g_lo = v_lo // _CW
        g_hi = v_hi // _CW
        n_ch = g_hi - g_lo + 1

        def fire(g, buf, sem):
            @pl.when(g < n_full)
            def _():
                off = pl.multiple_of(g * _CW, _CW)
                pltpu.make_async_copy(
                    table_hbm.at[:, pl.ds(off, _CW)],
                    staged_v.at[buf],
                    sem,
                ).start()

        def wait(g, buf, sem):
            @pl.when(g < n_full)
            def _():
                pltpu.make_async_copy(
                    table_hbm.at[:, pl.ds(0, _CW)], staged_v.at[buf], sem
                ).wait()

        def fire_any(g, k):
            @pl.when(lax.rem(k, 2) == 0)
            def _():
                fire(g, 0, sem0)

            @pl.when(lax.rem(k, 2) == 1)
            def _():
                fire(g, 1, sem1)

        def wait_any(g, k):
            @pl.when(lax.rem(k, 2) == 0)
            def _():
                wait(g, 0, sem0)

            @pl.when(lax.rem(k, 2) == 1)
            def _():
                wait(g, 1, sem1)

        fire_any(g_lo, 0)
        iota = lax.iota(jnp.int32, _L)

        def chunk_body(k, p):
            g = g_lo + k

            @pl.when(k + 1 < n_ch)
            def _():
                fire_any(g + 1, k + 1)

            wait_any(g, k)
            buf = lax.rem(k, 2)
            c_end = (g + 1) * _CW
            c_base = g * _CW
            buf_vec = jnp.full((_L,), buf, jnp.int32)
            is_tail = g >= n_full

            def hit_cond(p):
                win = sl_v[pl.ds(p, _L)]
                return (p < b_per_w) & (lax.reduce_min(win, (0,)) < c_end)

            def hit_body(p):
                win = sl_v[pl.ds(p, _L)]
                v_p = lax.reduce_min(win, (0,))
                p_vec = jnp.full((_L,), p, jnp.int32)

                @pl.when(jnp.logical_not(is_tail))
                def _():
                    c_loc = jnp.full((_L,), v_p - c_base, jnp.int32)
                    for j in range(D // _L):
                        d_vec = j * _L + iota
                        x = plsc.load_gather(staged_v,
                                             [buf_vec, d_vec, c_loc])
                        plsc.store_scatter(rows_v, [d_vec, p_vec], x)

                @pl.when(is_tail)
                def _():
                    c_loc = jnp.full((_L,), v_p - n_full * _CW, jnp.int32)
                    for j in range(D // _L):
                        d_vec = j * _L + iota
                        x = plsc.load_gather(tail_v, [d_vec * tail_w + c_loc])
                        plsc.store_scatter(rows_v, [d_vec, p_vec], x)

                return p + 1

            return lax.while_loop(hit_cond, hit_body, p)

        lax.fori_loop(0, n_ch, chunk_body, jnp.int32(0))
        pltpu.sync_copy(rows_v, out_hbm.at[:, pl.ds(base, b_per_w)])

    return stream_kernel(sorted_labels, table_t, table_tail)


def _ln_head_body(x_ref, g_ref, bt_ref, w_ref, bb_ref, y_ref, u_ref):
    x = x_ref[...]
    mean = jnp.mean(x, axis=0, keepdims=True)
    xc = x - mean
    var = jnp.mean(xc * xc, axis=0, keepdims=True)
    inv = lax.rsqrt(var + 1e-5)
    y = xc * inv * g_ref[...] + bt_ref[...]
    y_ref[...] = y
    z = jnp.sum(y * w_ref[...], axis=0, keepdims=True) + bb_ref[...]
    u_ref[...] = jax.nn.sigmoid(z)


def _tc_ln_head(rows_t, gamma, beta, W, b):
    D, B = rows_t.shape
    BLK = 2048
    grid = (B // BLK,)
    return pl.pallas_call(
        _ln_head_body,
        grid=grid,
        in_specs=[
            pl.BlockSpec((D, BLK), lambda i: (0, i)),
            pl.BlockSpec((D, 1), lambda i: (0, 0)),
            pl.BlockSpec((D, 1), lambda i: (0, 0)),
            pl.BlockSpec((D, 1), lambda i: (0, 0)),
            pl.BlockSpec((1, 1), lambda i: (0, 0)),
        ],
        out_specs=[
            pl.BlockSpec((D, BLK), lambda i: (0, i)),
            pl.BlockSpec((1, BLK), lambda i: (0, i)),
        ],
        out_shape=[
            jax.ShapeDtypeStruct((D, B), jnp.float32),
            jax.ShapeDtypeStruct((1, B), jnp.float32),
        ],
    )(rows_t, gamma.reshape(D, 1), beta.reshape(D, 1),
      W.reshape(D, 1), b.reshape(1, 1))


@jax.jit
def kernel(class_labels, emb_table, ln_gamma, ln_beta, unc_W, unc_b):
    B = class_labels.shape[0]
    labels = class_labels.astype(jnp.int32)
    sorted_labels, perm = lax.sort_key_val(
        labels, lax.iota(jnp.int32, B))
    V = emb_table.shape[0]
    n_full = V // _CW
    table_tail = emb_table[n_full * _CW:, :].T.reshape(-1)
    rows_t = _sc_stream_extract(sorted_labels, emb_table.T, table_tail)
    y_t, u_t = _tc_ln_head(rows_t, ln_gamma, ln_beta, unc_W, unc_b)
    y_s = y_t.T
    u_s = u_t.reshape(B, 1)
    y = jnp.zeros_like(y_s).at[perm].set(y_s)
    u = jnp.zeros_like(u_s).at[perm].set(u_s)
    return (y, u)


# speedup vs baseline: 1.5882x; 1.1180x over previous
import functools

import jax
import jax.numpy as jnp
from jax import lax
from jax.experimental import pallas as pl
from jax.experimental.pallas import tpu as pltpu
from jax.experimental.pallas import tpu_sc as plsc

_NW = 32
_L = 16
_CW = 256


def _sc_stream_fused(sorted_labels, perm2, params, table_t, table_tail):
    D, V = table_t.shape
    B = sorted_labels.shape[0]
    b_per_w = B // _NW
    n_full = V // _CW
    tail_w = V - n_full * _CW
    mesh = plsc.VectorSubcoreMesh(core_axis_name="c", subcore_axis_name="s")

    @functools.partial(
        pl.kernel,
        out_type=jax.ShapeDtypeStruct((B, 128), jnp.float32),
        mesh=mesh,
        compiler_params=pltpu.CompilerParams(needs_layout_passes=False),
        scratch_types=[
            pltpu.VMEM((b_per_w + 2 * _L,), jnp.int32),
            pltpu.VMEM((b_per_w // 128, 128), jnp.int32),
            pltpu.VMEM((3 * D + _L,), jnp.float32),
            pltpu.VMEM((2, D, _CW), jnp.float32),
            pltpu.VMEM((D * tail_w,), jnp.float32),
            pltpu.VMEM((b_per_w, 128), jnp.float32),
            pltpu.SemaphoreType.DMA,
            pltpu.SemaphoreType.DMA,
            pltpu.SemaphoreType.DMA,
        ],
    )
    def stream_kernel(sorted_hbm, perm_hbm, params_hbm, table_hbm, tail_hbm,
                      out_hbm, sl_v, perm_v, params_v, staged_v, tail_v,
                      rows_v, sem0, sem1, sem_out):
        wid = lax.axis_index("s") * 2 + lax.axis_index("c")
        base = wid * b_per_w
        pltpu.sync_copy(sorted_hbm.at[pl.ds(base, b_per_w)],
                        sl_v.at[pl.ds(0, b_per_w)])
        pltpu.sync_copy(perm_hbm.at[pl.ds(wid * (b_per_w // 128),
                                          b_per_w // 128)], perm_v)
        pltpu.sync_copy(params_hbm, params_v)
        pltpu.sync_copy(tail_hbm, tail_v)
        big = jnp.full((_L,), jnp.int32(2**30), jnp.int32)
        sl_v[pl.ds(b_per_w, _L)] = big
        sl_v[pl.ds(b_per_w + _L, _L)] = big

        v_lo = lax.reduce_min(sl_v[pl.ds(0, _L)], (0,))
        v_hi = lax.reduce_max(sl_v[pl.ds(b_per_w - _L, _L)], (0,))
        g_lo = v_lo // _CW
        g_hi = v_hi // _CW
        n_ch = g_hi - g_lo + 1

        def fire(g, buf, sem):
            @pl.when(g < n_full)
            def _():
                off = pl.multiple_of(g * _CW, _CW)
                pltpu.make_async_copy(
                    table_hbm.at[:, pl.ds(off, _CW)],
                    staged_v.at[buf],
                    sem,
                ).start()

        def wait(g, buf, sem):
            @pl.when(g < n_full)
            def _():
                pltpu.make_async_copy(
                    table_hbm.at[:, pl.ds(0, _CW)], staged_v.at[buf], sem
                ).wait()

        def fire_any(g, k):
            @pl.when(lax.rem(k, 2) == 0)
            def _():
                fire(g, 0, sem0)

            @pl.when(lax.rem(k, 2) == 1)
            def _():
                fire(g, 1, sem1)

        def wait_any(g, k):
            @pl.when(lax.rem(k, 2) == 0)
            def _():
                wait(g, 0, sem0)

            @pl.when(lax.rem(k, 2) == 1)
            def _():
                wait(g, 1, sem1)

        fire_any(g_lo, 0)
        iota = lax.iota(jnp.int32, _L)

        def chunk_body(k, p):
            g = g_lo + k

            @pl.when(k + 1 < n_ch)
            def _():
                fire_any(g + 1, k + 1)

            wait_any(g, k)
            buf = lax.rem(k, 2)
            c_end = (g + 1) * _CW
            c_base = g * _CW
            buf_vec = jnp.full((_L,), buf, jnp.int32)
            is_tail = g >= n_full

            def hit_cond(p):
                win = sl_v[pl.ds(p, _L)]
                return (p < b_per_w) & (lax.reduce_min(win, (0,)) < c_end)

            def hit_body(p):
                win = sl_v[pl.ds(p, _L)]
                v_p = lax.reduce_min(win, (0,))
                p_vec = jnp.full((_L,), p, jnp.int32)

                @pl.when(jnp.logical_not(is_tail))
                def _():
                    c_loc = jnp.full((_L,), v_p - c_base, jnp.int32)
                    for j in range(D // _L):
                        d_vec = j * _L + iota
                        x = plsc.load_gather(staged_v,
                                             [buf_vec, d_vec, c_loc])
                        plsc.store_scatter(rows_v, [p_vec, d_vec], x)

                @pl.when(is_tail)
                def _():
                    c_loc = jnp.full((_L,), v_p - n_full * _CW, jnp.int32)
                    for j in range(D // _L):
                        d_vec = j * _L + iota
                        x = plsc.load_gather(tail_v, [d_vec * tail_w + c_loc])
                        plsc.store_scatter(rows_v, [p_vec, d_vec], x)

                return p + 1

            return lax.while_loop(hit_cond, hit_body, p)

        lax.fori_loop(0, n_ch, chunk_body, jnp.int32(0))

        g_regs = [params_v[pl.ds(j * _L, _L)] for j in range(D // _L)]
        b_regs = [params_v[pl.ds(D + j * _L, _L)] for j in range(D // _L)]
        w_regs = [params_v[pl.ds(2 * D + j * _L, _L)] for j in range(D // _L)]
        bias_vec = params_v[pl.ds(3 * D, _L)]
        inv_d = jnp.float32(1.0 / D)

        def ln_body(grp, carry):
            p_vec = grp * _L + iota
            s = jnp.zeros((_L,), jnp.float32)
            s2 = jnp.zeros((_L,), jnp.float32)
            for d in range(D):
                d_vec = jnp.full((_L,), d, jnp.int32)
                x = plsc.load_gather(rows_v, [p_vec, d_vec])
                s = s + x
                s2 = s2 + x * x
            mean = s * inv_d
            var = s2 * inv_d - mean * mean
            ve = var + jnp.float32(1e-5)
            yr = plsc.bitcast(
                jnp.int32(0x5F3759DF) - (plsc.bitcast(ve, jnp.int32) >> 1),
                jnp.float32)
            for _ in range(3):
                yr = yr * (jnp.float32(1.5)
                           - jnp.float32(0.5) * ve * yr * yr)
            z = jnp.zeros((_L,), jnp.float32)
            for d in range(D):
                d_vec = jnp.full((_L,), d, jnp.int32)
                x = plsc.load_gather(rows_v, [p_vec, d_vec])
                gd = jnp.full((_L,), g_regs[d // _L][d % _L], jnp.float32)
                bd = jnp.full((_L,), b_regs[d // _L][d % _L], jnp.float32)
                wd = jnp.full((_L,), w_regs[d // _L][d % _L], jnp.float32)
                y = (x - mean) * yr * gd + bd
                plsc.store_scatter(rows_v, [p_vec, d_vec], y)
                z = z + y * wd
            z = z + bias_vec
            u = jnp.float32(1.0) / (jnp.float32(1.0) + jnp.exp(-z))
            plsc.store_scatter(rows_v, [p_vec, jnp.full((_L,), D, jnp.int32)],
                               u)
            return carry

        lax.fori_loop(0, b_per_w // _L, ln_body, jnp.int32(0))

        copies = []
        for j in range(b_per_w // 128):
            copies.append(pltpu.make_async_copy(
                rows_v.at[pl.ds(j * 128, 128)],
                out_hbm.at[perm_v.at[j]],
                sem_out,
            ))
            copies[-1].start()
        for c in copies:
            c.wait()

    return stream_kernel(sorted_labels, perm2, params, table_t, table_tail)


@jax.jit
def kernel(class_labels, emb_table, ln_gamma, ln_beta, unc_W, unc_b):
    B = class_labels.shape[0]
    V, D = emb_table.shape
    labels = class_labels.astype(jnp.int32)
    sorted_labels, perm = lax.sort_key_val(labels, lax.iota(jnp.int32, B))
    perm2 = perm.reshape(B // 128, 128)
    n_full = V // _CW
    table_tail = emb_table[n_full * _CW:, :].T.reshape(-1)
    params = jnp.concatenate([
        ln_gamma, ln_beta, unc_W.reshape(-1),
        jnp.broadcast_to(unc_b.reshape(1), (_L,)),
    ]).astype(jnp.float32)
    out = _sc_stream_fused(sorted_labels, perm2, params,
                           emb_table.T, table_tail)
    y = out[:, :D]
    u = out[:, D:D + 1]
    return (y, u)


# speedup vs baseline: 1.7982x; 1.1323x over previous
import functools

import jax
import jax.numpy as jnp
from jax import lax
from jax.experimental import pallas as pl
from jax.experimental.pallas import tpu as pltpu
from jax.experimental.pallas import tpu_sc as plsc

_NW = 32
_L = 16
_CW = 256


def _sc_stream_fused(sorted_labels, perm2, params, table_t, table_tail):
    D, V = table_t.shape
    B = sorted_labels.shape[0]
    b_per_w = B // _NW
    n_full = V // _CW
    tail_w = V - n_full * _CW
    mesh = plsc.VectorSubcoreMesh(core_axis_name="c", subcore_axis_name="s")

    @functools.partial(
        pl.kernel,
        out_type=jax.ShapeDtypeStruct((B, 128), jnp.float32),
        mesh=mesh,
        compiler_params=pltpu.CompilerParams(needs_layout_passes=False),
        scratch_types=[
            pltpu.VMEM((b_per_w + 2 * _L,), jnp.int32),
            pltpu.VMEM((b_per_w // 128, 128), jnp.int32),
            pltpu.VMEM((3 * D + _L,), jnp.float32),
            pltpu.VMEM((2, D, _CW), jnp.float32),
            pltpu.VMEM((D * tail_w,), jnp.float32),
            pltpu.VMEM((b_per_w, 128), jnp.float32),
            pltpu.SemaphoreType.DMA,
            pltpu.SemaphoreType.DMA,
            pltpu.SemaphoreType.DMA,
        ],
    )
    def stream_kernel(sorted_hbm, perm_hbm, params_hbm, table_hbm, tail_hbm,
                      out_hbm, sl_v, perm_v, params_v, staged_v, tail_v,
                      rows_v, sem0, sem1, sem_out):
        wid = lax.axis_index("s") * 2 + lax.axis_index("c")
        base = wid * b_per_w
        pltpu.sync_copy(sorted_hbm.at[pl.ds(base, b_per_w)],
                        sl_v.at[pl.ds(0, b_per_w)])
        pltpu.sync_copy(perm_hbm.at[pl.ds(wid * (b_per_w // 128),
                                          b_per_w // 128)], perm_v)
        pltpu.sync_copy(params_hbm, params_v)
        pltpu.sync_copy(tail_hbm, tail_v)
        big = jnp.full((_L,), jnp.int32(2**30), jnp.int32)
        sl_v[pl.ds(b_per_w, _L)] = big
        sl_v[pl.ds(b_per_w + _L, _L)] = big

        v_lo = lax.reduce_min(sl_v[pl.ds(0, _L)], (0,))
        v_hi = lax.reduce_max(sl_v[pl.ds(b_per_w - _L, _L)], (0,))
        g_lo = v_lo // _CW
        g_hi = v_hi // _CW
        n_ch = g_hi - g_lo + 1

        def fire(g, buf, sem):
            @pl.when(g < n_full)
            def _():
                off = pl.multiple_of(g * _CW, _CW)
                pltpu.make_async_copy(
                    table_hbm.at[:, pl.ds(off, _CW)],
                    staged_v.at[buf],
                    sem,
                ).start()

        def wait(g, buf, sem):
            @pl.when(g < n_full)
            def _():
                pltpu.make_async_copy(
                    table_hbm.at[:, pl.ds(0, _CW)], staged_v.at[buf], sem
                ).wait()

        def fire_any(g, k):
            @pl.when(lax.rem(k, 2) == 0)
            def _():
                fire(g, 0, sem0)

            @pl.when(lax.rem(k, 2) == 1)
            def _():
                fire(g, 1, sem1)

        def wait_any(g, k):
            @pl.when(lax.rem(k, 2) == 0)
            def _():
                wait(g, 0, sem0)

            @pl.when(lax.rem(k, 2) == 1)
            def _():
                wait(g, 1, sem1)

        fire_any(g_lo, 0)
        iota = lax.iota(jnp.int32, _L)

        def chunk_body(k, p):
            g = g_lo + k

            @pl.when(k + 1 < n_ch)
            def _():
                fire_any(g + 1, k + 1)

            wait_any(g, k)
            buf = lax.rem(k, 2)
            c_end = (g + 1) * _CW
            c_base = g * _CW
            buf_vec = jnp.full((_L,), buf, jnp.int32)
            is_tail = g >= n_full

            def hit_cond(p):
                win = sl_v[pl.ds(p, _L)]
                return (p < b_per_w) & (lax.reduce_min(win, (0,)) < c_end)

            def hit_body(p):
                win = sl_v[pl.ds(p, _L)]
                v_p = lax.reduce_min(win, (0,))
                p_vec = jnp.full((_L,), p, jnp.int32)

                @pl.when(jnp.logical_not(is_tail))
                def _():
                    c_loc = jnp.full((_L,), v_p - c_base, jnp.int32)
                    for j in range(D // _L):
                        d_vec = j * _L + iota
                        x = plsc.load_gather(staged_v,
                                             [buf_vec, d_vec, c_loc])
                        plsc.store_scatter(rows_v, [p_vec, d_vec], x)

                @pl.when(is_tail)
                def _():
                    c_loc = jnp.full((_L,), v_p - n_full * _CW, jnp.int32)
                    for j in range(D // _L):
                        d_vec = j * _L + iota
                        x = plsc.load_gather(tail_v, [d_vec * tail_w + c_loc])
                        plsc.store_scatter(rows_v, [p_vec, d_vec], x)

                return p + 1

            return lax.while_loop(hit_cond, hit_body, p)

        lax.fori_loop(0, n_ch, chunk_body, jnp.int32(0))

        g_regs = [params_v[pl.ds(j * _L, _L)] for j in range(D // _L)]
        b_regs = [params_v[pl.ds(D + j * _L, _L)] for j in range(D // _L)]
        w_regs = [params_v[pl.ds(2 * D + j * _L, _L)] for j in range(D // _L)]
        bias_vec = params_v[pl.ds(3 * D, _L)]
        inv_d = jnp.float32(1.0 / D)

        def ln_body(grp, carry):
            p_vec = grp * _L + iota
            xs = []
            for d in range(D):
                d_vec = jnp.full((_L,), d, jnp.int32)
                xs.append(plsc.load_gather(rows_v, [p_vec, d_vec]))
            acc = [jnp.zeros((_L,), jnp.float32) for _ in range(4)]
            acc2 = [jnp.zeros((_L,), jnp.float32) for _ in range(4)]
            for d in range(D):
                acc[d % 4] = acc[d % 4] + xs[d]
                acc2[d % 4] = acc2[d % 4] + xs[d] * xs[d]
            s = (acc[0] + acc[1]) + (acc[2] + acc[3])
            s2 = (acc2[0] + acc2[1]) + (acc2[2] + acc2[3])
            mean = s * inv_d
            var = s2 * inv_d - mean * mean
            ve = var + jnp.float32(1e-5)
            yr = plsc.bitcast(
                jnp.int32(0x5F3759DF) - (plsc.bitcast(ve, jnp.int32) >> 1),
                jnp.float32)
            for _ in range(3):
                yr = yr * (jnp.float32(1.5)
                           - jnp.float32(0.5) * ve * yr * yr)
            zacc = [jnp.zeros((_L,), jnp.float32) for _ in range(4)]
            for d in range(D):
                d_vec = jnp.full((_L,), d, jnp.int32)
                gd = jnp.full((_L,), g_regs[d // _L][d % _L], jnp.float32)
                bd = jnp.full((_L,), b_regs[d // _L][d % _L], jnp.float32)
                wd = jnp.full((_L,), w_regs[d // _L][d % _L], jnp.float32)
                y = (xs[d] - mean) * yr * gd + bd
                plsc.store_scatter(rows_v, [p_vec, d_vec], y)
                zacc[d % 4] = zacc[d % 4] + y * wd
            z = (zacc[0] + zacc[1]) + (zacc[2] + zacc[3]) + bias_vec
            u = jnp.float32(1.0) / (jnp.float32(1.0) + jnp.exp(-z))
            plsc.store_scatter(rows_v, [p_vec, jnp.full((_L,), D, jnp.int32)],
                               u)
            return carry

        lax.fori_loop(0, b_per_w // _L, ln_body, jnp.int32(0))

        copies = []
        for j in range(b_per_w // 128):
            copies.append(pltpu.make_async_copy(
                rows_v.at[pl.ds(j * 128, 128)],
                out_hbm.at[perm_v.at[j]],
                sem_out,
            ))
            copies[-1].start()
        for c in copies:
            c.wait()

    return stream_kernel(sorted_labels, perm2, params, table_t, table_tail)


@jax.jit
def kernel(class_labels, emb_table, ln_gamma, ln_beta, unc_W, unc_b):
    B = class_labels.shape[0]
    V, D = emb_table.shape
    labels = class_labels.astype(jnp.int32)
    sorted_labels, perm = lax.sort_key_val(labels, lax.iota(jnp.int32, B))
    perm2 = perm.reshape(B // 128, 128)
    n_full = V // _CW
    table_tail = emb_table[n_full * _CW:, :].T.reshape(-1)
    params = jnp.concatenate([
        ln_gamma, ln_beta, unc_W.reshape(-1),
        jnp.broadcast_to(unc_b.reshape(1), (_L,)),
    ]).astype(jnp.float32)
    out = _sc_stream_fused(sorted_labels, perm2, params,
                           emb_table.T, table_tail)
    y = out[:, :D]
    u = out[:, D:D + 1]
    return (y, u)


# speedup vs baseline: 1.8356x; 1.0208x over previous
import functools

import jax
import jax.numpy as jnp
from jax import lax
from jax.experimental import pallas as pl
from jax.experimental.pallas import tpu as pltpu
from jax.experimental.pallas import tpu_sc as plsc

_NW = 32
_L = 16
_CW = 256


def _sc_stream_fused(sorted_labels, perm2, params, table_t, table_tail):
    D, V = table_t.shape
    B = sorted_labels.shape[0]
    b_per_w = B // _NW
    n_full = V // _CW
    tail_w = V - n_full * _CW
    mesh = plsc.VectorSubcoreMesh(core_axis_name="c", subcore_axis_name="s")

    @functools.partial(
        pl.kernel,
        out_type=jax.ShapeDtypeStruct((B, 128), jnp.float32),
        mesh=mesh,
        compiler_params=pltpu.CompilerParams(needs_layout_passes=False),
        scratch_types=[
            pltpu.VMEM((b_per_w + 2 * _L,), jnp.int32),
            pltpu.VMEM((b_per_w // 128, 128), jnp.int32),
            pltpu.VMEM((3 * D + _L,), jnp.float32),
            pltpu.VMEM((2, D, _CW), jnp.float32),
            pltpu.VMEM((D * tail_w,), jnp.float32),
            pltpu.VMEM((b_per_w, 128), jnp.float32),
            pltpu.SemaphoreType.DMA,
            pltpu.SemaphoreType.DMA,
            pltpu.SemaphoreType.DMA,
        ],
    )
    def stream_kernel(sorted_hbm, perm_hbm, params_hbm, table_hbm, tail_hbm,
                      out_hbm, sl_v, perm_v, params_v, staged_v, tail_v,
                      rows_v, sem0, sem1, sem_out):
        wid = lax.axis_index("s") * 2 + lax.axis_index("c")
        base = wid * b_per_w
        pltpu.sync_copy(sorted_hbm.at[pl.ds(base, b_per_w)],
                        sl_v.at[pl.ds(0, b_per_w)])
        pltpu.sync_copy(perm_hbm.at[pl.ds(wid * (b_per_w // 128),
                                          b_per_w // 128)], perm_v)
        pltpu.sync_copy(params_hbm, params_v)
        pltpu.sync_copy(tail_hbm, tail_v)
        big = jnp.full((_L,), jnp.int32(2**30), jnp.int32)
        sl_v[pl.ds(b_per_w, _L)] = big
        sl_v[pl.ds(b_per_w + _L, _L)] = big

        v_lo = lax.reduce_min(sl_v[pl.ds(0, _L)], (0,))
        v_hi = lax.reduce_max(sl_v[pl.ds(b_per_w - _L, _L)], (0,))
        g_lo = v_lo // _CW
        g_hi = v_hi // _CW
        n_ch = g_hi - g_lo + 1

        def fire(g, buf, sem):
            @pl.when(g < n_full)
            def _():
                off = pl.multiple_of(g * _CW, _CW)
                pltpu.make_async_copy(
                    table_hbm.at[:, pl.ds(off, _CW)],
                    staged_v.at[buf],
                    sem,
                ).start()

        def wait(g, buf, sem):
            @pl.when(g < n_full)
            def _():
                pltpu.make_async_copy(
                    table_hbm.at[:, pl.ds(0, _CW)], staged_v.at[buf], sem
                ).wait()

        def fire_any(g, k):
            @pl.when(lax.rem(k, 2) == 0)
            def _():
                fire(g, 0, sem0)

            @pl.when(lax.rem(k, 2) == 1)
            def _():
                fire(g, 1, sem1)

        def wait_any(g, k):
            @pl.when(lax.rem(k, 2) == 0)
            def _():
                wait(g, 0, sem0)

            @pl.when(lax.rem(k, 2) == 1)
            def _():
                wait(g, 1, sem1)

        fire_any(g_lo, 0)
        iota = lax.iota(jnp.int32, _L)

        g_regs = [params_v[pl.ds(j * _L, _L)] for j in range(D // _L)]
        b_regs = [params_v[pl.ds(D + j * _L, _L)] for j in range(D // _L)]
        w_regs = [params_v[pl.ds(2 * D + j * _L, _L)] for j in range(D // _L)]
        bias_vec = params_v[pl.ds(3 * D, _L)]
        inv_d = jnp.float32(1.0 / D)

        def ln_group(grp):
            p_vec = grp * _L + iota
            xs = []
            for d in range(D):
                d_vec = jnp.full((_L,), d, jnp.int32)
                xs.append(plsc.load_gather(rows_v, [p_vec, d_vec]))
            acc = [jnp.zeros((_L,), jnp.float32) for _ in range(4)]
            acc2 = [jnp.zeros((_L,), jnp.float32) for _ in range(4)]
            for d in range(D):
                acc[d % 4] = acc[d % 4] + xs[d]
                acc2[d % 4] = acc2[d % 4] + xs[d] * xs[d]
            s = (acc[0] + acc[1]) + (acc[2] + acc[3])
            s2 = (acc2[0] + acc2[1]) + (acc2[2] + acc2[3])
            mean = s * inv_d
            var = s2 * inv_d - mean * mean
            ve = var + jnp.float32(1e-5)
            yr = plsc.bitcast(
                jnp.int32(0x5F3759DF) - (plsc.bitcast(ve, jnp.int32) >> 1),
                jnp.float32)
            for _ in range(3):
                yr = yr * (jnp.float32(1.5)
                           - jnp.float32(0.5) * ve * yr * yr)
            zacc = [jnp.zeros((_L,), jnp.float32) for _ in range(4)]
            for d in range(D):
                d_vec = jnp.full((_L,), d, jnp.int32)
                gd = jnp.full((_L,), g_regs[d // _L][d % _L], jnp.float32)
                bd = jnp.full((_L,), b_regs[d // _L][d % _L], jnp.float32)
                wd = jnp.full((_L,), w_regs[d // _L][d % _L], jnp.float32)
                y = (xs[d] - mean) * yr * gd + bd
                plsc.store_scatter(rows_v, [p_vec, d_vec], y)
                zacc[d % 4] = zacc[d % 4] + y * wd
            z = (zacc[0] + zacc[1]) + (zacc[2] + zacc[3]) + bias_vec
            u = jnp.float32(1.0) / (jnp.float32(1.0) + jnp.exp(-z))
            plsc.store_scatter(rows_v, [p_vec, jnp.full((_L,), D, jnp.int32)],
                               u)

        def chunk_body(k, carry):
            p, q = carry
            g = g_lo + k

            @pl.when(k + 1 < n_ch)
            def _():
                fire_any(g + 1, k + 1)

            wait_any(g, k)
            buf = lax.rem(k, 2)
            c_end = (g + 1) * _CW
            c_base = g * _CW
            buf_vec = jnp.full((_L,), buf, jnp.int32)
            is_tail = g >= n_full

            def hit_cond(p):
                win = sl_v[pl.ds(p, _L)]
                return (p < b_per_w) & (lax.reduce_min(win, (0,)) < c_end)

            def hit_body(p):
                win = sl_v[pl.ds(p, _L)]
                v_p = lax.reduce_min(win, (0,))
                p_vec = jnp.full((_L,), p, jnp.int32)

                @pl.when(jnp.logical_not(is_tail))
                def _():
                    c_loc = jnp.full((_L,), v_p - c_base, jnp.int32)
                    for j in range(D // _L):
                        d_vec = j * _L + iota
                        x = plsc.load_gather(staged_v,
                                             [buf_vec, d_vec, c_loc])
                        plsc.store_scatter(rows_v, [p_vec, d_vec], x)

                @pl.when(is_tail)
                def _():
                    c_loc = jnp.full((_L,), v_p - n_full * _CW, jnp.int32)
                    for j in range(D // _L):
                        d_vec = j * _L + iota
                        x = plsc.load_gather(tail_v, [d_vec * tail_w + c_loc])
                        plsc.store_scatter(rows_v, [p_vec, d_vec], x)

                return p + 1

            p = lax.while_loop(hit_cond, hit_body, p)

            def ln_cond(q):
                return (q + 1) * _L <= p

            def ln_step(q):
                ln_group(q)
                return q + 1

            q = lax.while_loop(ln_cond, ln_step, q)
            return (p, q)

        _, q_fin = lax.fori_loop(0, n_ch, chunk_body,
                                 (jnp.int32(0), jnp.int32(0)))

        def ln_cond2(q):
            return q < b_per_w // _L

        def ln_step2(q):
            ln_group(q)
            return q + 1

        lax.while_loop(ln_cond2, ln_step2, q_fin)

        copies = []
        for j in range(b_per_w // 128):
            copies.append(pltpu.make_async_copy(
                rows_v.at[pl.ds(j * 128, 128)],
                out_hbm.at[perm_v.at[j]],
                sem_out,
            ))
            copies[-1].start()
        for c in copies:
            c.wait()

    return stream_kernel(sorted_labels, perm2, params, table_t, table_tail)


@jax.jit
def kernel(class_labels, emb_table, ln_gamma, ln_beta, unc_W, unc_b):
    B = class_labels.shape[0]
    V, D = emb_table.shape
    labels = class_labels.astype(jnp.int32)
    sorted_labels, perm = lax.sort_key_val(labels, lax.iota(jnp.int32, B))
    perm2 = perm.reshape(B // 128, 128)
    n_full = V // _CW
    table_tail = emb_table[n_full * _CW:, :].T.reshape(-1)
    params = jnp.concatenate([
        ln_gamma, ln_beta, unc_W.reshape(-1),
        jnp.broadcast_to(unc_b.reshape(1), (_L,)),
    ]).astype(jnp.float32)
    out = _sc_stream_fused(sorted_labels, perm2, params,
                           emb_table.T, table_tail)
    y = out[:, :D]
    u = out[:, D:D + 1]
    return (y, u)


# speedup vs baseline: 1.9273x; 1.0499x over previous
import functools

import jax
import jax.numpy as jnp
from jax import lax
from jax.experimental import pallas as pl
from jax.experimental.pallas import tpu as pltpu
from jax.experimental.pallas import tpu_sc as plsc

_NW = 32
_L = 16
_CW = 384


def _sc_stream_fused(sorted_labels, perm2, params, table_t, table_tail):
    D, V = table_t.shape
    B = sorted_labels.shape[0]
    b_per_w = B // _NW
    n_full = V // _CW
    tail_w = V - n_full * _CW
    mesh = plsc.VectorSubcoreMesh(core_axis_name="c", subcore_axis_name="s")

    @functools.partial(
        pl.kernel,
        out_type=jax.ShapeDtypeStruct((B, 128), jnp.float32),
        mesh=mesh,
        compiler_params=pltpu.CompilerParams(needs_layout_passes=False),
        scratch_types=[
            pltpu.VMEM((b_per_w + 2 * _L,), jnp.int32),
            pltpu.VMEM((b_per_w // 128, 128), jnp.int32),
            pltpu.VMEM((3 * D + _L,), jnp.float32),
            pltpu.VMEM((2, D, _CW), jnp.float32),
            pltpu.VMEM((D * tail_w,), jnp.float32),
            pltpu.VMEM((b_per_w, 128), jnp.float32),
            pltpu.SemaphoreType.DMA,
            pltpu.SemaphoreType.DMA,
            pltpu.SemaphoreType.DMA,
        ],
    )
    def stream_kernel(sorted_hbm, perm_hbm, params_hbm, table_hbm, tail_hbm,
                      out_hbm, sl_v, perm_v, params_v, staged_v, tail_v,
                      rows_v, sem0, sem1, sem_out):
        wid = lax.axis_index("s") * 2 + lax.axis_index("c")
        base = wid * b_per_w
        pltpu.sync_copy(sorted_hbm.at[pl.ds(base, b_per_w)],
                        sl_v.at[pl.ds(0, b_per_w)])
        pltpu.sync_copy(perm_hbm.at[pl.ds(wid * (b_per_w // 128),
                                          b_per_w // 128)], perm_v)
        pltpu.sync_copy(params_hbm, params_v)
        pltpu.sync_copy(tail_hbm, tail_v)
        big = jnp.full((_L,), jnp.int32(2**30), jnp.int32)
        sl_v[pl.ds(b_per_w, _L)] = big
        sl_v[pl.ds(b_per_w + _L, _L)] = big

        v_lo = lax.reduce_min(sl_v[pl.ds(0, _L)], (0,))
        v_hi = lax.reduce_max(sl_v[pl.ds(b_per_w - _L, _L)], (0,))
        g_lo = v_lo // _CW
        g_hi = v_hi // _CW
        n_ch = g_hi - g_lo + 1

        def fire(g, buf, sem):
            @pl.when(g < n_full)
            def _():
                off = pl.multiple_of(g * _CW, _CW)
                pltpu.make_async_copy(
                    table_hbm.at[:, pl.ds(off, _CW)],
                    staged_v.at[buf],
                    sem,
                ).start()

        def wait(g, buf, sem):
            @pl.when(g < n_full)
            def _():
                pltpu.make_async_copy(
                    table_hbm.at[:, pl.ds(0, _CW)], staged_v.at[buf], sem
                ).wait()

        def fire_any(g, k):
            @pl.when(lax.rem(k, 2) == 0)
            def _():
                fire(g, 0, sem0)

            @pl.when(lax.rem(k, 2) == 1)
            def _():
                fire(g, 1, sem1)

        def wait_any(g, k):
            @pl.when(lax.rem(k, 2) == 0)
            def _():
                wait(g, 0, sem0)

            @pl.when(lax.rem(k, 2) == 1)
            def _():
                wait(g, 1, sem1)

        fire_any(g_lo, 0)
        iota = lax.iota(jnp.int32, _L)

        g_regs = [params_v[pl.ds(j * _L, _L)] for j in range(D // _L)]
        b_regs = [params_v[pl.ds(D + j * _L, _L)] for j in range(D // _L)]
        w_regs = [params_v[pl.ds(2 * D + j * _L, _L)] for j in range(D // _L)]
        bias_vec = params_v[pl.ds(3 * D, _L)]
        inv_d = jnp.float32(1.0 / D)

        def ln_group(grp):
            p_vec = grp * _L + iota
            xs = []
            for d in range(D):
                d_vec = jnp.full((_L,), d, jnp.int32)
                xs.append(plsc.load_gather(rows_v, [p_vec, d_vec]))
            acc = [jnp.zeros((_L,), jnp.float32) for _ in range(4)]
            acc2 = [jnp.zeros((_L,), jnp.float32) for _ in range(4)]
            for d in range(D):
                acc[d % 4] = acc[d % 4] + xs[d]
                acc2[d % 4] = acc2[d % 4] + xs[d] * xs[d]
            s = (acc[0] + acc[1]) + (acc[2] + acc[3])
            s2 = (acc2[0] + acc2[1]) + (acc2[2] + acc2[3])
            mean = s * inv_d
            var = s2 * inv_d - mean * mean
            ve = var + jnp.float32(1e-5)
            yr = plsc.bitcast(
                jnp.int32(0x5F3759DF) - (plsc.bitcast(ve, jnp.int32) >> 1),
                jnp.float32)
            for _ in range(3):
                yr = yr * (jnp.float32(1.5)
                           - jnp.float32(0.5) * ve * yr * yr)
            zacc = [jnp.zeros((_L,), jnp.float32) for _ in range(4)]
            for d in range(D):
                d_vec = jnp.full((_L,), d, jnp.int32)
                gd = jnp.full((_L,), g_regs[d // _L][d % _L], jnp.float32)
                bd = jnp.full((_L,), b_regs[d // _L][d % _L], jnp.float32)
                wd = jnp.full((_L,), w_regs[d // _L][d % _L], jnp.float32)
                y = (xs[d] - mean) * yr * gd + bd
                plsc.store_scatter(rows_v, [p_vec, d_vec], y)
                zacc[d % 4] = zacc[d % 4] + y * wd
            z = (zacc[0] + zacc[1]) + (zacc[2] + zacc[3]) + bias_vec
            u = jnp.float32(1.0) / (jnp.float32(1.0) + jnp.exp(-z))
            plsc.store_scatter(rows_v, [p_vec, jnp.full((_L,), D, jnp.int32)],
                               u)

        def chunk_body(k, carry):
            p, q = carry
            g = g_lo + k

            @pl.when(k + 1 < n_ch)
            def _():
                fire_any(g + 1, k + 1)

            wait_any(g, k)
            buf = lax.rem(k, 2)
            c_end = (g + 1) * _CW
            c_base = g * _CW
            buf_vec = jnp.full((_L,), buf, jnp.int32)
            is_tail = g >= n_full

            def hit_cond(p):
                win = sl_v[pl.ds(p, _L)]
                return (p < b_per_w) & (lax.reduce_min(win, (0,)) < c_end)

            def hit_body(p):
                win = sl_v[pl.ds(p, _L)]
                v_p = lax.reduce_min(win, (0,))
                p_vec = jnp.full((_L,), p, jnp.int32)

                @pl.when(jnp.logical_not(is_tail))
                def _():
                    c_loc = jnp.full((_L,), v_p - c_base, jnp.int32)
                    for j in range(D // _L):
                        d_vec = j * _L + iota
                        x = plsc.load_gather(staged_v,
                                             [buf_vec, d_vec, c_loc])
                        plsc.store_scatter(rows_v, [p_vec, d_vec], x)

                @pl.when(is_tail)
                def _():
                    c_loc = jnp.full((_L,), v_p - n_full * _CW, jnp.int32)
                    for j in range(D // _L):
                        d_vec = j * _L + iota
                        x = plsc.load_gather(tail_v, [d_vec * tail_w + c_loc])
                        plsc.store_scatter(rows_v, [p_vec, d_vec], x)

                return p + 1

            p = lax.while_loop(hit_cond, hit_body, p)

            def ln_cond(q):
                return (q + 1) * _L <= p

            def ln_step(q):
                ln_group(q)
                return q + 1

            q = lax.while_loop(ln_cond, ln_step, q)
            return (p, q)

        _, q_fin = lax.fori_loop(0, n_ch, chunk_body,
                                 (jnp.int32(0), jnp.int32(0)))

        def ln_cond2(q):
            return q < b_per_w // _L

        def ln_step2(q):
            ln_group(q)
            return q + 1

        lax.while_loop(ln_cond2, ln_step2, q_fin)

        copies = []
        for j in range(b_per_w // 128):
            copies.append(pltpu.make_async_copy(
                rows_v.at[pl.ds(j * 128, 128)],
                out_hbm.at[perm_v.at[j]],
                sem_out,
            ))
            copies[-1].start()
        for c in copies:
            c.wait()

    return stream_kernel(sorted_labels, perm2, params, table_t, table_tail)


@jax.jit
def kernel(class_labels, emb_table, ln_gamma, ln_beta, unc_W, unc_b):
    B = class_labels.shape[0]
    V, D = emb_table.shape
    labels = class_labels.astype(jnp.int32)
    sorted_labels, perm = lax.sort_key_val(labels, lax.iota(jnp.int32, B))
    perm2 = perm.reshape(B // 128, 128)
    n_full = V // _CW
    table_tail = emb_table[n_full * _CW:, :].T.reshape(-1)
    params = jnp.concatenate([
        ln_gamma, ln_beta, unc_W.reshape(-1),
        jnp.broadcast_to(unc_b.reshape(1), (_L,)),
    ]).astype(jnp.float32)
    out = _sc_stream_fused(sorted_labels, perm2, params,
                           emb_table.T, table_tail)
    y = out[:, :D]
    u = out[:, D:D + 1]
    return (y, u)


# speedup vs baseline: 1.9353x; 1.0042x over previous
import functools

import jax
import jax.numpy as jnp
from jax import lax
from jax.experimental import pallas as pl
from jax.experimental.pallas import tpu as pltpu
from jax.experimental.pallas import tpu_sc as plsc

_NW = 32
_L = 16
_CW = 384


def _sc_stream_fused(sorted_labels, perm2, params, table_t, table_tail):
    D, V = table_t.shape
    B = sorted_labels.shape[0]
    b_per_w = B // _NW
    n_full = V // _CW
    tail_w = V - n_full * _CW
    mesh = plsc.VectorSubcoreMesh(core_axis_name="c", subcore_axis_name="s")

    @functools.partial(
        pl.kernel,
        out_type=jax.ShapeDtypeStruct((B, 128), jnp.float32),
        mesh=mesh,
        compiler_params=pltpu.CompilerParams(needs_layout_passes=False),
        scratch_types=[
            pltpu.VMEM((b_per_w + 2 * _L,), jnp.int32),
            pltpu.VMEM((b_per_w // 128, 128), jnp.int32),
            pltpu.VMEM((3 * D + _L,), jnp.float32),
            pltpu.VMEM((2, D, _CW), jnp.float32),
            pltpu.VMEM((D * tail_w,), jnp.float32),
            pltpu.VMEM((b_per_w, 128), jnp.float32),
            pltpu.SemaphoreType.DMA,
            pltpu.SemaphoreType.DMA,
            pltpu.SemaphoreType.DMA,
        ],
    )
    def stream_kernel(sorted_hbm, perm_hbm, params_hbm, table_hbm, tail_hbm,
                      out_hbm, sl_v, perm_v, params_v, staged_v, tail_v,
                      rows_v, sem0, sem1, sem_out):
        wid = lax.axis_index("s") * 2 + lax.axis_index("c")
        base = wid * b_per_w
        pltpu.sync_copy(sorted_hbm.at[pl.ds(base, b_per_w)],
                        sl_v.at[pl.ds(0, b_per_w)])
        pltpu.sync_copy(perm_hbm.at[pl.ds(wid * (b_per_w // 128),
                                          b_per_w // 128)], perm_v)
        pltpu.sync_copy(params_hbm, params_v)
        pltpu.sync_copy(tail_hbm, tail_v)
        big = jnp.full((_L,), jnp.int32(2**30), jnp.int32)
        sl_v[pl.ds(b_per_w, _L)] = big
        sl_v[pl.ds(b_per_w + _L, _L)] = big

        v_lo = lax.reduce_min(sl_v[pl.ds(0, _L)], (0,))
        v_hi = lax.reduce_max(sl_v[pl.ds(b_per_w - _L, _L)], (0,))
        g_lo = v_lo // _CW
        g_hi = v_hi // _CW
        n_ch = g_hi - g_lo + 1

        def fire(g, buf, sem):
            @pl.when(g < n_full)
            def _():
                off = pl.multiple_of(g * _CW, _CW)
                pltpu.make_async_copy(
                    table_hbm.at[:, pl.ds(off, _CW)],
                    staged_v.at[buf],
                    sem,
                ).start()

        def wait(g, buf, sem):
            @pl.when(g < n_full)
            def _():
                pltpu.make_async_copy(
                    table_hbm.at[:, pl.ds(0, _CW)], staged_v.at[buf], sem
                ).wait()

        def fire_any(g, k):
            @pl.when(lax.rem(k, 2) == 0)
            def _():
                fire(g, 0, sem0)

            @pl.when(lax.rem(k, 2) == 1)
            def _():
                fire(g, 1, sem1)

        def wait_any(g, k):
            @pl.when(lax.rem(k, 2) == 0)
            def _():
                wait(g, 0, sem0)

            @pl.when(lax.rem(k, 2) == 1)
            def _():
                wait(g, 1, sem1)

        fire_any(g_lo, 0)
        iota = lax.iota(jnp.int32, _L)

        g_regs = [params_v[pl.ds(j * _L, _L)] for j in range(D // _L)]
        b_regs = [params_v[pl.ds(D + j * _L, _L)] for j in range(D // _L)]
        w_regs = [params_v[pl.ds(2 * D + j * _L, _L)] for j in range(D // _L)]
        bias_vec = params_v[pl.ds(3 * D, _L)]
        inv_d = jnp.float32(1.0 / D)

        def ln_group(grp):
            p_vec = grp * _L + iota
            xs = []
            for d in range(D):
                d_vec = jnp.full((_L,), d, jnp.int32)
                xs.append(plsc.load_gather(rows_v, [p_vec, d_vec]))
            acc = [jnp.zeros((_L,), jnp.float32) for _ in range(4)]
            acc2 = [jnp.zeros((_L,), jnp.float32) for _ in range(4)]
            for d in range(D):
                acc[d % 4] = acc[d % 4] + xs[d]
                acc2[d % 4] = acc2[d % 4] + xs[d] * xs[d]
            s = (acc[0] + acc[1]) + (acc[2] + acc[3])
            s2 = (acc2[0] + acc2[1]) + (acc2[2] + acc2[3])
            mean = s * inv_d
            var = s2 * inv_d - mean * mean
            ve = var + jnp.float32(1e-5)
            yr = plsc.bitcast(
                jnp.int32(0x5F3759DF) - (plsc.bitcast(ve, jnp.int32) >> 1),
                jnp.float32)
            for _ in range(3):
                yr = yr * (jnp.float32(1.5)
                           - jnp.float32(0.5) * ve * yr * yr)
            zacc = [jnp.zeros((_L,), jnp.float32) for _ in range(4)]
            for d in range(D):
                d_vec = jnp.full((_L,), d, jnp.int32)
                gd = jnp.full((_L,), g_regs[d // _L][d % _L], jnp.float32)
                bd = jnp.full((_L,), b_regs[d // _L][d % _L], jnp.float32)
                wd = jnp.full((_L,), w_regs[d // _L][d % _L], jnp.float32)
                y = (xs[d] - mean) * yr * gd + bd
                plsc.store_scatter(rows_v, [p_vec, d_vec], y)
                zacc[d % 4] = zacc[d % 4] + y * wd
            z = (zacc[0] + zacc[1]) + (zacc[2] + zacc[3]) + bias_vec
            u = jnp.float32(1.0) / (jnp.float32(1.0) + jnp.exp(-z))
            plsc.store_scatter(rows_v, [p_vec, jnp.full((_L,), D, jnp.int32)],
                               u)

        def chunk_body(k, carry):
            p, q = carry
            g = g_lo + k

            @pl.when(k + 1 < n_ch)
            def _():
                fire_any(g + 1, k + 1)

            wait_any(g, k)
            buf = lax.rem(k, 2)
            c_end = (g + 1) * _CW
            c_base = g * _CW
            buf_vec = jnp.full((_L,), buf, jnp.int32)
            is_tail = g >= n_full

            def hit_cond(p):
                win = sl_v[pl.ds(p, _L)]
                return (p < b_per_w) & (win[0] < c_end)

            def hit_body(p):
                win = sl_v[pl.ds(p, _L)]
                v_p = win[0]
                p_vec = jnp.full((_L,), p, jnp.int32)

                @pl.when(jnp.logical_not(is_tail))
                def _():
                    c_loc = jnp.full((_L,), v_p - c_base, jnp.int32)
                    for j in range(D // _L):
                        d_vec = j * _L + iota
                        x = plsc.load_gather(staged_v,
                                             [buf_vec, d_vec, c_loc])
                        plsc.store_scatter(rows_v, [p_vec, d_vec], x)

                @pl.when(is_tail)
                def _():
                    c_loc = jnp.full((_L,), v_p - n_full * _CW, jnp.int32)
                    for j in range(D // _L):
                        d_vec = j * _L + iota
                        x = plsc.load_gather(tail_v, [d_vec * tail_w + c_loc])
                        plsc.store_scatter(rows_v, [p_vec, d_vec], x)

                return p + 1

            p = lax.while_loop(hit_cond, hit_body, p)

            def ln_cond(q):
                return (q + 1) * _L <= p

            def ln_step(q):
                ln_group(q)
                return q + 1

            q = lax.while_loop(ln_cond, ln_step, q)
            return (p, q)

        _, q_fin = lax.fori_loop(0, n_ch, chunk_body,
                                 (jnp.int32(0), jnp.int32(0)))

        def ln_cond2(q):
            return q < b_per_w // _L

        def ln_step2(q):
            ln_group(q)
            return q + 1

        lax.while_loop(ln_cond2, ln_step2, q_fin)

        copies = []
        for j in range(b_per_w // 128):
            copies.append(pltpu.make_async_copy(
                rows_v.at[pl.ds(j * 128, 128)],
                out_hbm.at[perm_v.at[j]],
                sem_out,
            ))
            copies[-1].start()
        for c in copies:
            c.wait()

    return stream_kernel(sorted_labels, perm2, params, table_t, table_tail)


@jax.jit
def kernel(class_labels, emb_table, ln_gamma, ln_beta, unc_W, unc_b):
    B = class_labels.shape[0]
    V, D = emb_table.shape
    labels = class_labels.astype(jnp.int32)
    sorted_labels, perm = lax.sort_key_val(labels, lax.iota(jnp.int32, B))
    perm2 = perm.reshape(B // 128, 128)
    n_full = V // _CW
    table_tail = emb_table[n_full * _CW:, :].T.reshape(-1)
    params = jnp.concatenate([
        ln_gamma, ln_beta, unc_W.reshape(-1),
        jnp.broadcast_to(unc_b.reshape(1), (_L,)),
    ]).astype(jnp.float32)
    out = _sc_stream_fused(sorted_labels, perm2, params,
                           emb_table.T, table_tail)
    y = out[:, :D]
    u = out[:, D:D + 1]
    return (y, u)
